# Initial kernel scaffold; baseline (speedup 1.0000x reference)
import jax
import jax.numpy as jnp
from jax.experimental import pallas as pl


def kernel(x, xi, xe, y, yi, ye, g1_Wl, g1_bl, g1_Wr, g1_br, g1_We, g1_att, g1_bias, g1_Wxz, g1_Whz, g1_Wxr, g1_Whr, g1_Wxh, g1_Whh, g1_bxz, g1_bhz, g1_bxr, g1_bhr, g1_bxh, g1_bhh, g1_W1, g1_b1, g1_gamma, g1_beta, g1_W2, g1_b2, g2_Wl, g2_bl, g2_Wr, g2_br, g2_We, g2_att, g2_bias, g2_Wxz, g2_Whz, g2_Wxr, g2_Whr, g2_Wxh, g2_Whh, g2_bxz, g2_bhz, g2_bxr, g2_bhr, g2_bxh, g2_bhh, g2_W1, g2_b1, g2_gamma, g2_beta, g2_W2, g2_b2, cW1, cb1, cW2, cb2):
    raise NotImplementedError("write your pallas kernel here")


# speedup vs baseline: 1.6455x; 1.6455x over previous
import jax
import jax.numpy as jnp
from jax.experimental import pallas as pl


def _leaky(x):
    return jnp.where(x >= 0, x, 0.2 * x)


def _gatv2(x, ei, ea, p):
    n = x.shape[0]
    src = ei[0]
    dst = ei[1]
    xl = x @ p['Wl'] + p['bl']
    xr = x @ p['Wr'] + p['br']
    cnt = jax.ops.segment_sum(jnp.ones((ea.shape[0],), jnp.float32), dst, num_segments=n)
    loop = jax.ops.segment_sum(ea, dst, num_segments=n) / jnp.maximum(cnt, 1.0)[:, None]
    ar = jnp.arange(n)
    s = jnp.concatenate([src, ar])
    d = jnp.concatenate([dst, ar])
    eaa = jnp.concatenate([ea, loop], 0)
    m = _leaky(xl[s] + xr[d] + eaa @ p['We'])
    lg = m @ p['att']
    ex = jnp.exp(lg)
    den = jax.ops.segment_sum(ex, d, num_segments=n)
    al = ex / (den[d] + 1e-16)
    out = jax.ops.segment_sum(al[:, None] * xl[s], d, num_segments=n)
    return out + p['bias']


def _cheb_ts(x, src, dst, dis, n):
    w = -(dis[src] * dis[dst])

    def lh(v):
        return jax.ops.segment_sum(w[:, None] * v[src], dst, num_segments=n)

    t0 = x
    t1 = lh(t0)
    t2 = 2.0 * lh(t1) - t0
    t3 = 2.0 * lh(t2) - t1
    return t0, t1, t2, t3


def _gru(x, ei, p, n):
    src = ei[0]
    dst = ei[1]
    deg = jax.ops.segment_sum(jnp.ones((src.shape[0],), jnp.float32), src, num_segments=n)
    dis = jnp.where(deg > 0, deg ** -0.5, 0.0)
    t0, t1, t2, t3 = _cheb_ts(x, src, dst, dis, n)
    T = jnp.concatenate([t0, t1, t2, t3], axis=1)
    Wz = jnp.concatenate([p['Wxz'][k] for k in range(4)], axis=0)
    Wh = jnp.concatenate([p['Wxh'][k] for k in range(4)], axis=0)
    Z = jax.nn.sigmoid(T @ Wz + p['bxz'] + p['bhz'])
    Ht = jnp.tanh(T @ Wh + p['bxh'] + p['bhh'])
    return (1.0 - Z) * Ht


def _mlpaggr_mean(x, p):
    n = x.shape[0]
    h = x @ p['W1'][:, :16].T + p['b1']
    s1 = h.sum(0) + p['b1']
    s2 = (h * h).sum(0) + p['b1'] * p['b1']
    mu = s1 / (n + 1)
    var = s2 / (n + 1) - mu * mu
    inv = 1.0 / jnp.sqrt(var + 1e-5)
    hn = (h - mu) * inv * p['gamma'] + p['beta']
    h0 = (p['b1'] - mu) * inv * p['gamma'] + p['beta']
    rmean = (jax.nn.relu(hn).sum(0) + jax.nn.relu(h0)) / (n + 1)
    return rmean @ p['W2'].T + p['b2']


def _branch_mean(x, ei, ea, p):
    o = _gatv2(x, ei, ea, p)
    o = _gru(o, ei, p, x.shape[0])
    return _mlpaggr_mean(o, p)


def _head_kernel(d1_ref, d2_ref, w1_ref, b1_ref, w2_ref, b2_ref, o_ref):
    dvec = d1_ref[0, :] - d2_ref[0, :]
    h = jnp.maximum(dvec @ w1_ref[...].T + b1_ref[0, :], 0.0)
    o_ref[...] = (jnp.sum(h * w2_ref[0, :]) + b2_ref[0, 0]).reshape(1, 1)


def kernel(x, xi, xe, y, yi, ye, g1_Wl, g1_bl, g1_Wr, g1_br, g1_We, g1_att, g1_bias, g1_Wxz, g1_Whz, g1_Wxr, g1_Whr, g1_Wxh, g1_Whh, g1_bxz, g1_bhz, g1_bxr, g1_bhr, g1_bxh, g1_bhh, g1_W1, g1_b1, g1_gamma, g1_beta, g1_W2, g1_b2, g2_Wl, g2_bl, g2_Wr, g2_br, g2_We, g2_att, g2_bias, g2_Wxz, g2_Whz, g2_Wxr, g2_Whr, g2_Wxh, g2_Whh, g2_bxz, g2_bhz, g2_bxr, g2_bhr, g2_bxh, g2_bhh, g2_W1, g2_b1, g2_gamma, g2_beta, g2_W2, g2_b2, cW1, cb1, cW2, cb2):
    inp = dict(locals())
    p1 = {k[3:]: v for k, v in inp.items() if k.startswith('g1_')}
    p2 = {k[3:]: v for k, v in inp.items() if k.startswith('g2_')}
    d1 = _branch_mean(x, xi, xe, p1)
    d2 = _branch_mean(y, yi, ye, p2)
    out = pl.pallas_call(
        _head_kernel,
        out_shape=jax.ShapeDtypeStruct((1, 1), jnp.float32),
    )(d1.reshape(1, 8), d2.reshape(1, 8), cW1, cb1.reshape(1, 16),
      cW2, cb2.reshape(1, 1))
    return out.reshape(1)


# speedup vs baseline: 13.6835x; 8.3157x over previous
import dataclasses
import functools

import jax
import jax.numpy as jnp
from jax import lax
from jax.experimental import pallas as pl
from jax.experimental.pallas import tpu as pltpu
from jax.experimental.pallas import tpu_sc as plsc

N = 10000
E = 160000
NS = 16
EPT = E // NS
NPT = 640
NPAD = NS * NPT
NPK = NPAD // 16
NPKT = NPK // NS
CH = 80
RB = 1280
TROWS = 8 * NPAD + NPK

_SC_MESH = dict(core_axis_name="c", subcore_axis_name="s")


def _sc_params():
    cp = pltpu.CompilerParams()
    fields = pltpu.CompilerParams.__dataclass_fields__
    kw = {}
    if "needs_layout_passes" in fields:
        kw["needs_layout_passes"] = False
    if "use_tc_tiling_on_sc" in fields:
        kw["use_tc_tiling_on_sc"] = False
    return dataclasses.replace(cp, **kw)


def _leaky(v):
    return jnp.where(v >= 0, v, 0.2 * v)


def _bcast1(buf, i):
    return plsc.load_gather(buf, [jnp.full((16,), i, jnp.int32)])


def _sc_graph(ei_flat, xl_cat, xr_cat, eaw_cat, att2, gbias2):
    mesh = plsc.VectorSubcoreMesh(**_SC_MESH)

    @functools.partial(
        pl.kernel,
        out_type=jax.ShapeDtypeStruct((TROWS, 16), jnp.float32),
        mesh=mesh,
        compiler_params=_sc_params(),
        scratch_types=[
            pltpu.VMEM((EPT,), jnp.int32),
            pltpu.VMEM((EPT,), jnp.int32),
            pltpu.VMEM((EPT,), jnp.float32),
            pltpu.VMEM((NPK, 16), jnp.float32),
            pltpu.VMEM((NPK, 16), jnp.float32),
            pltpu.VMEM((NPK, 16), jnp.float32),
            pltpu.VMEM((NPK, 16), jnp.float32),
            pltpu.VMEM((CH,), jnp.int32),
            pltpu.VMEM((CH,), jnp.int32),
            pltpu.VMEM((CH,), jnp.float32),
            pltpu.VMEM((CH,), jnp.float32),
            pltpu.VMEM((CH, 16), jnp.float32),
            pltpu.VMEM((CH, 16), jnp.float32),
            pltpu.VMEM((CH, 16), jnp.float32),
            pltpu.VMEM((NPT, 16), jnp.float32),
            pltpu.VMEM((NPKT, 16), jnp.float32),
            pltpu.VMEM((NPKT, 16), jnp.float32),
            pltpu.VMEM((NPKT, 16), jnp.float32),
            pltpu.VMEM_SHARED((NPAD, 16), jnp.float32),
            pltpu.VMEM_SHARED((NPAD, 16), jnp.float32),
            pltpu.VMEM_SHARED((NPK, 16), jnp.float32),
            pltpu.VMEM_SHARED((NPK, 16), jnp.float32),
            pltpu.VMEM_SHARED((NPK, 16), jnp.float32),
        ],
    )
    def k(ei_hbm, xl_hbm, xr_hbm, eaw_hbm, att_hbm, gb_hbm, t_hbm,
          src_v, dst_v, wch_v, cnt2_v, deg2_v, den2_v, dis2_v,
          gi_v, gl_v, wc_v, iv_v, rowA_v, rowB_v, rowC_v,
          own_v, cntO_v, denO_v, disO_v,
          S_p, S_acc, S_cnt, S_deg, S_den):
        c = lax.axis_index("c")
        s = lax.axis_index("s")
        zeros16 = jnp.zeros((16,), jnp.float32)
        ones16 = jnp.ones((16,), jnp.float32)
        iota16 = jnp.arange(16, dtype=jnp.int32)
        nbase = s * NPT

        pltpu.sync_copy(ei_hbm.at[pl.ds((2 * c + 0) * E + s * EPT, EPT)],
                        dst_v)
        pltpu.sync_copy(ei_hbm.at[pl.ds((2 * c + 1) * E + s * EPT, EPT)],
                        src_v)

        @pl.loop(0, CH, step=16)
        def _(j):
            gi_v[pl.ds(j, 16)] = jnp.full((16,), 0, jnp.int32) + c

        pltpu.sync_copy(att_hbm.at[gi_v], rowA_v)
        pltpu.sync_copy(gb_hbm.at[gi_v], rowB_v)
        attv = rowA_v[0, :]
        gbias = rowB_v[0, :]

        @pl.loop(0, NPK)
        def _(r):
            cnt2_v[r, :] = zeros16
            deg2_v[r, :] = zeros16
            den2_v[r, :] = zeros16

        @pl.loop(0, CH)
        def _(i):
            rowC_v[i, :] = zeros16

        @pl.loop(0, NPT, step=CH)
        def _(i):
            pltpu.sync_copy(rowC_v, S_p.at[pl.ds(nbase + i, CH), :])
            pltpu.sync_copy(rowC_v, S_acc.at[pl.ds(nbase + i, CH), :])

        pltpu.sync_copy(rowC_v.at[pl.ds(0, NPKT), :],
                        S_cnt.at[pl.ds(s * NPKT, NPKT), :])
        pltpu.sync_copy(rowC_v.at[pl.ds(0, NPKT), :],
                        S_deg.at[pl.ds(s * NPKT, NPKT), :])
        pltpu.sync_copy(rowC_v.at[pl.ds(0, NPKT), :],
                        S_den.at[pl.ds(s * NPKT, NPKT), :])
        plsc.subcore_barrier()

        @pl.loop(0, EPT, step=CH)
        def _(i):
            @pl.loop(0, CH, step=16)
            def _(j):
                gi_v[pl.ds(j, 16)] = src_v[pl.ds(i + j, 16)] + c * NPAD

            pltpu.sync_copy(xl_hbm.at[gi_v], rowA_v)

            @pl.loop(0, CH, step=16)
            def _(j):
                gi_v[pl.ds(j, 16)] = iota16 + (c * E + s * EPT + i + j)

            pltpu.sync_copy(eaw_hbm.at[gi_v], rowC_v)

            @pl.loop(0, CH, step=16)
            def _(j):
                dd = dst_v[pl.ds(i + j, 16)]
                gl_v[pl.ds(j, 16)] = dd
                gi_v[pl.ds(j, 16)] = dd + c * NPAD

            pltpu.sync_copy(xr_hbm.at[gi_v], rowB_v)
            pltpu.sync_copy(rowC_v, S_p.at[gl_v], add=True)

            @pl.loop(0, CH, step=16)
            def _(j):
                dd = dst_v[pl.ds(i + j, 16)]
                ss = src_v[pl.ds(i + j, 16)]
                plsc.addupdate_scatter(
                    cnt2_v, [dd >> 4, dd & 15], ones16)
                plsc.addupdate_scatter(
                    deg2_v, [ss >> 4, ss & 15], ones16)

            @pl.loop(0, CH)
            def _(e):
                msum = rowA_v[e, :] + rowB_v[e, :] + rowC_v[e, :]
                rowC_v[e, :] = _leaky(msum) * attv

            @pl.loop(0, CH, step=16)
            def _(j):
                acc = zeros16
                for f in range(16):
                    acc = acc + plsc.load_gather(
                        rowC_v, [iota16 + j, jnp.full((16,), f, jnp.int32)])
                w = jnp.exp(acc)
                wc_v[pl.ds(j, 16)] = w
                dd = dst_v[pl.ds(i + j, 16)]
                plsc.addupdate_scatter(den2_v, [dd >> 4, dd & 15], w)

            @pl.loop(0, CH)
            def _(e):
                rowB_v[e, :] = rowA_v[e, :] * _bcast1(wc_v, e)

            pltpu.sync_copy(rowB_v, S_acc.at[gl_v], add=True)

        for part, acc in ((cnt2_v, S_cnt), (deg2_v, S_deg), (den2_v, S_den)):
            @pl.loop(0, NPK, step=CH)
            def _(i):
                @pl.loop(0, CH, step=16)
                def _(j):
                    gi_v[pl.ds(j, 16)] = iota16 + (i + j)

                @pl.loop(0, CH)
                def _(e):
                    rowC_v[e, :] = part[i + e, :]

                pltpu.sync_copy(rowC_v, acc.at[gi_v], add=True)

        plsc.subcore_barrier()

        pltpu.sync_copy(S_cnt.at[pl.ds(s * NPKT, NPKT), :], cntO_v)
        pltpu.sync_copy(S_den.at[pl.ds(s * NPKT, NPKT), :], denO_v)
        pltpu.sync_copy(S_deg.at[pl.ds(s * NPKT, NPKT), :], disO_v)

        @pl.loop(0, NPKT)
        def _(r):
            d = disO_v[r, :]
            bits = plsc.bitcast(d, jnp.int32)
            g = plsc.bitcast(jnp.full((16,), 0x5F3759DF, jnp.int32)
                             - (bits >> 1), jnp.float32)
            for _unused in range(3):
                g = g * (1.5 - 0.5 * d * g * g)
            disO_v[r, :] = jnp.where(d > 0, g, 0.0)

        pltpu.sync_copy(disO_v,
                        t_hbm.at[pl.ds(8 * NPAD + s * NPKT, NPKT), :])

        @pl.loop(0, NPT, step=CH)
        def _(b):
            pltpu.sync_copy(xl_hbm.at[pl.ds(c * NPAD + nbase + b, CH), :],
                            rowA_v)
            pltpu.sync_copy(xr_hbm.at[pl.ds(c * NPAD + nbase + b, CH), :],
                            rowB_v)
            pltpu.sync_copy(S_p.at[pl.ds(nbase + b, CH), :], rowC_v)

            @pl.loop(0, CH, step=16)
            def _(j):
                cnt16 = cntO_v[(b + j) >> 4, :]
                iv_v[pl.ds(j, 16)] = 1.0 / jnp.maximum(cnt16, 1.0)

            @pl.loop(0, CH)
            def _(e):
                msum = (rowA_v[e, :] + rowB_v[e, :]
                        + rowC_v[e, :] * _bcast1(iv_v, e))
                rowC_v[e, :] = _leaky(msum) * attv

            @pl.loop(0, CH, step=16)
            def _(j):
                acc = zeros16
                for f in range(16):
                    acc = acc + plsc.load_gather(
                        rowC_v, [iota16 + j, jnp.full((16,), f, jnp.int32)])
                w = jnp.exp(acc)
                wc_v[pl.ds(j, 16)] = w
                dtot = denO_v[(b + j) >> 4, :] + w
                iv_v[pl.ds(j, 16)] = 1.0 / (dtot + 1e-16)

            pltpu.sync_copy(S_acc.at[pl.ds(nbase + b, CH), :], rowB_v)

            @pl.loop(0, CH)
            def _(e):
                numr = rowB_v[e, :] + rowA_v[e, :] * _bcast1(wc_v, e)
                own_v[b + e, :] = numr * _bcast1(iv_v, e) + gbias

        @pl.loop(0, CH)
        def _(i):
            rowA_v[i, :] = zeros16

        @pl.loop(0, NPT, step=CH)
        def _(i):
            pltpu.sync_copy(rowA_v, S_acc.at[pl.ds(nbase + i, CH), :])

        pltpu.sync_copy(own_v,
                        t_hbm.at[pl.ds(4 * c * NPAD + nbase, NPT), :])
        plsc.subcore_barrier()

        pltpu.sync_copy(t_hbm.at[pl.ds(8 * NPAD, NPK), :], dis2_v)

        @pl.loop(0, EPT, step=16)
        def _(i):
            sv = src_v[pl.ds(i, 16)]
            dv = dst_v[pl.ds(i, 16)]
            a = plsc.load_gather(dis2_v, [sv >> 4, sv & 15])
            bb = plsc.load_gather(dis2_v, [dv >> 4, dv & 15])
            wch_v[pl.ds(i, 16)] = -(a * bb)

        def cheb_scatter(kprev):
            @pl.loop(0, EPT, step=CH)
            def _(i):
                @pl.loop(0, CH, step=16)
                def _(j):
                    gi_v[pl.ds(j, 16)] = (src_v[pl.ds(i + j, 16)]
                                          + (4 * c + kprev) * NPAD)
                    gl_v[pl.ds(j, 16)] = dst_v[pl.ds(i + j, 16)]

                pltpu.sync_copy(t_hbm.at[gi_v], rowA_v)

                @pl.loop(0, CH)
                def _(e):
                    rowB_v[e, :] = rowA_v[e, :] * _bcast1(wch_v, i + e)

                pltpu.sync_copy(rowB_v, S_acc.at[gl_v], add=True)

        def cheb_update(k, coef):
            @pl.loop(0, NPT, step=CH)
            def _(b):
                pltpu.sync_copy(S_acc.at[pl.ds(nbase + b, CH), :], rowA_v)
                if k == 1:
                    @pl.loop(0, CH)
                    def _(e):
                        own_v[b + e, :] = rowA_v[e, :]
                else:
                    pltpu.sync_copy(
                        t_hbm.at[pl.ds((4 * c + k - 2) * NPAD + nbase + b,
                                       CH), :], rowB_v)

                    @pl.loop(0, CH)
                    def _(e):
                        own_v[b + e, :] = coef * rowA_v[e, :] - rowB_v[e, :]

            pltpu.sync_copy(
                own_v, t_hbm.at[pl.ds((4 * c + k) * NPAD + nbase, NPT), :])

        for k, coef in ((1, 1.0), (2, 2.0), (3, 2.0)):
            cheb_scatter(k - 1)
            plsc.subcore_barrier()
            cheb_update(k, coef)
            if k < 3:
                @pl.loop(0, CH)
                def _(i):
                    rowA_v[i, :] = zeros16

                @pl.loop(0, NPT, step=CH)
                def _(i):
                    pltpu.sync_copy(rowA_v,
                                    S_acc.at[pl.ds(nbase + i, CH), :])

                plsc.subcore_barrier()

    return k(ei_flat, xl_cat, xr_cat, eaw_cat, att2, gbias2)


def _proj_kernel(x_ref, w_ref, b_ref, o_ref):
    o_ref[0] = jnp.dot(x_ref[0], w_ref[0],
                       preferred_element_type=jnp.float32,
                       precision=lax.Precision.HIGHEST) + b_ref[0, 0]


def _tc_proj(xy_pad, Wlr, blr):
    return pl.pallas_call(
        _proj_kernel,
        grid=(2, NPAD // RB),
        in_specs=[
            pl.BlockSpec((1, RB, 256), lambda b, i: (b, i, 0)),
            pl.BlockSpec((1, 256, 32), lambda b, i: (b, 0, 0)),
            pl.BlockSpec((1, 1, 32), lambda b, i: (b, 0, 0)),
        ],
        out_specs=pl.BlockSpec((1, RB, 32), lambda b, i: (b, i, 0)),
        out_shape=jax.ShapeDtypeStruct((2, NPAD, 32), jnp.float32),
    )(xy_pad, Wlr, blr)


def _eaw_kernel(ea_ref, w_ref, o_ref):
    o_ref[0] = jnp.dot(ea_ref[0], w_ref[0],
                       preferred_element_type=jnp.float32,
                       precision=lax.Precision.HIGHEST)


def _tc_eaw(ea_pack, Wblk):
    EB = E // 8 // 5
    return pl.pallas_call(
        _eaw_kernel,
        grid=(2, 5),
        in_specs=[
            pl.BlockSpec((1, EB, 128), lambda b, i: (b, i, 0)),
            pl.BlockSpec((1, 128, 128), lambda b, i: (b, 0, 0)),
        ],
        out_specs=pl.BlockSpec((1, EB, 128), lambda b, i: (b, i, 0)),
        out_shape=jax.ShapeDtypeStruct((2, E // 8, 128), jnp.float32),
    )(ea_pack, Wblk)


def _tail_kernel(tt_ref, wz_ref, wh_ref, bz_ref, bh_ref,
                 w1_ref, b1_ref, gm_ref, bt_ref, w2_ref, b2_ref,
                 cw1_ref, cb1_ref, cw2_ref, cb2_ref, o_ref):
    hi = lax.Precision.HIGHEST
    o8 = []
    for c in range(2):
        ttc = tt_ref[c]
        zlin = jnp.dot(wz_ref[c], ttc, preferred_element_type=jnp.float32,
                       precision=hi) + bz_ref[c][:, None]
        hlin = jnp.dot(wh_ref[c], ttc, preferred_element_type=jnp.float32,
                       precision=hi) + bh_ref[c][:, None]
        g = (1.0 - jax.nn.sigmoid(zlin)) * jnp.tanh(hlin)
        b1 = b1_ref[c]
        h4 = jnp.dot(w1_ref[c], g, preferred_element_type=jnp.float32,
                     precision=hi) + b1[:, None]
        s1 = jnp.sum(h4, axis=1) + b1
        s2 = jnp.sum(h4 * h4, axis=1) + b1 * b1
        mu = s1 / (N + 1)
        var = s2 / (N + 1) - mu * mu
        inv = 1.0 / jnp.sqrt(var + 1e-5)
        gm = gm_ref[c]
        bt = bt_ref[c]
        hn = (h4 - mu[:, None]) * (inv * gm)[:, None] + bt[:, None]
        h0 = (b1 - mu) * inv * gm + bt
        rmean = (jnp.sum(jnp.maximum(hn, 0.0), axis=1)
                 + jnp.maximum(h0, 0.0)) / (N + 1)
        o8.append(jnp.dot(w2_ref[c], rmean.reshape(4, 1),
                          preferred_element_type=jnp.float32,
                          precision=hi)[:, 0] + b2_ref[c])
    dvec = o8[0] - o8[1]
    h = jnp.maximum(jnp.dot(cw1_ref[...], dvec.reshape(8, 1),
                            preferred_element_type=jnp.float32,
                            precision=hi)[:, 0] + cb1_ref[0], 0.0)
    o_ref[...] = (jnp.sum(h * cw2_ref[0, :]) + cb2_ref[0, 0]).reshape(1, 1)


def _tc_tail(TT, Wz, Wh, bz, bh, W1s, b1s, gms, bts, W2s, b2s,
             cW1, cb1, cW2, cb2):
    return pl.pallas_call(
        _tail_kernel,
        out_shape=jax.ShapeDtypeStruct((1, 1), jnp.float32),
    )(TT, Wz, Wh, bz, bh, W1s, b1s, gms, bts, W2s, b2s,
      cW1, cb1.reshape(1, 16), cW2, cb2.reshape(1, 1))


def kernel(x, xi, xe, y, yi, ye, g1_Wl, g1_bl, g1_Wr, g1_br, g1_We, g1_att, g1_bias, g1_Wxz, g1_Whz, g1_Wxr, g1_Whr, g1_Wxh, g1_Whh, g1_bxz, g1_bhz, g1_bxr, g1_bhr, g1_bxh, g1_bhh, g1_W1, g1_b1, g1_gamma, g1_beta, g1_W2, g1_b2, g2_Wl, g2_bl, g2_Wr, g2_br, g2_We, g2_att, g2_bias, g2_Wxz, g2_Whz, g2_Wxr, g2_Whr, g2_Wxh, g2_Whh, g2_bxz, g2_bhz, g2_bxr, g2_bhr, g2_bxh, g2_bhh, g2_W1, g2_b1, g2_gamma, g2_beta, g2_W2, g2_b2, cW1, cb1, cW2, cb2):
    f32 = jnp.float32
    pad = ((0, NPAD - N), (0, 0))
    xy_pad = jnp.stack([jnp.pad(x, pad), jnp.pad(y, pad)])
    Wlr = jnp.stack([jnp.concatenate([g1_Wl, g1_Wr], 1),
                     jnp.concatenate([g2_Wl, g2_Wr], 1)])
    blr = jnp.stack([jnp.concatenate([g1_bl, g1_br]),
                     jnp.concatenate([g2_bl, g2_br])]).reshape(2, 1, 32)
    ea_pack = jnp.stack([xe.reshape(E // 8, 128), ye.reshape(E // 8, 128)])
    eye8 = jnp.eye(8, dtype=f32)
    Wblk = jnp.stack([jnp.kron(eye8, g1_We), jnp.kron(eye8, g2_We)])
    ei_flat = jnp.concatenate(
        [xi[1], xi[0], yi[1], yi[0]]).astype(jnp.int32)
    att2 = jnp.stack([g1_att, g2_att])
    gbias2 = jnp.stack([g1_bias, g2_bias])

    xlr = _tc_proj(xy_pad, Wlr, blr)
    xl_cat = xlr[:, :, :16].reshape(2 * NPAD, 16)
    xr_cat = xlr[:, :, 16:].reshape(2 * NPAD, 16)
    eaw_cat = _tc_eaw(ea_pack, Wblk).reshape(2 * E, 16)

    T = _sc_graph(ei_flat, xl_cat, xr_cat, eaw_cat, att2, gbias2)
    T4 = T[:8 * NPAD].reshape(2, 4, NPAD, 16)[:, :, :N, :]
    TT = jnp.transpose(T4, (0, 1, 3, 2)).reshape(2, 64, N)

    Wz = jnp.stack([jnp.concatenate(list(g1_Wxz), 0).T,
                    jnp.concatenate(list(g2_Wxz), 0).T])
    Wh = jnp.stack([jnp.concatenate(list(g1_Wxh), 0).T,
                    jnp.concatenate(list(g2_Wxh), 0).T])
    bz = jnp.stack([g1_bxz + g1_bhz, g2_bxz + g2_bhz])
    bh = jnp.stack([g1_bxh + g1_bhh, g2_bxh + g2_bhh])
    W1s = jnp.stack([g1_W1[:, :16], g2_W1[:, :16]])
    b1s = jnp.stack([g1_b1, g2_b1])
    gms = jnp.stack([g1_gamma, g2_gamma])
    bts = jnp.stack([g1_beta, g2_beta])
    W2s = jnp.stack([g1_W2, g2_W2])
    b2s = jnp.stack([g1_b2, g2_b2])
    out = _tc_tail(TT, Wz, Wh, bz, bh, W1s, b1s, gms, bts, W2s, b2s,
                   cW1, cb1, cW2, cb2)
    return out.reshape(1)
